# baseline (speedup 1.0000x reference)
import functools

import jax
import jax.numpy as jnp
import numpy as np
from jax import lax
from jax.experimental import pallas as pl
from jax.experimental.pallas import tpu as pltpu
from jax.experimental.pallas import tpu_sc as plsc

_N = 4096
_T = _N * (_N + 1) // 2
_IC = _N - 64
_BUF = 4224
_NW = 32


def _off(i):
    return i * _N - (i * (i - 1)) // 2


def _tail_constants():
    off_ic = _off(_IC)
    d_qt = off_ic & 7
    gt = off_ic - d_qt
    tail_len = _T - gt
    idx = np.zeros(2112, dtype=np.int32)
    i = _IC
    for s in range(tail_len):
        p = gt + s
        if p < off_ic:
            idx[s] = 4096 + 8 - d_qt + s
        else:
            while _off(i + 1) <= p:
                i += 1
            idx[s] = 64 * (i - _IC) + (64 - (_N - i)) + (p - _off(i))
    return gt, tail_len, idx


_GT, _TAIL_LEN, _TAIL_IDX = _tail_constants()


def _m8(v):
    return pl.multiple_of(v, 8)


def _geom(i):
    L = _N - i
    f0 = i * _N + i
    d_s = f0 & 7
    off = i * _N - ((i * (i - 1)) >> 1)
    end = off + L
    d_q = off & 7
    d_e = end & 7
    return dict(
        i=i, L=L, d_s=d_s, fa0=f0 - d_s, off=off, end=end,
        d_q=d_q, qa=off - d_q, ea=end - d_e, sh=8 + d_s - d_q,
    )


def _triu_body(x_hbm, tidx_hbm, out_hbm, gb0, gb1, sb0, sb1, tbuf, ibuf,
               sgA, sgB, ss0, ss1):
    NC = 2
    wid = lax.axis_index("s") * NC + lax.axis_index("c")
    lanes = lax.iota(jnp.int32, 16)

    def g_copies(g, C, gb, sem):
        i = g["i"]
        yield None, pltpu.make_async_copy(
            x_hbm.at[pl.ds(_m8(lax.max(i * _N - 8, 0)), 8)], gb.at[pl.ds(0, 8)], sem
        )
        yield None, pltpu.make_async_copy(
            x_hbm.at[pl.ds(_m8(g["fa0"]), C)], gb.at[pl.ds(8, C)], sem
        )
        yield g["L"] + g["d_s"] > 2 * C, pltpu.make_async_copy(
            x_hbm.at[pl.ds(_m8(g["fa0"] + C), C)], gb.at[pl.ds(8 + C, C)], sem
        )
        yield None, pltpu.make_async_copy(
            x_hbm.at[pl.ds(_m8(i * _N + _N - C), C)],
            gb.at[pl.ds(_m8(8 + g["L"] + g["d_s"] - C), C)],
            sem,
        )

    def s_copies(g, C, sb, sem):
        qa, ea = g["qa"], g["ea"]
        yield None, pltpu.make_async_copy(
            sb.at[pl.ds(0, C)], out_hbm.at[pl.ds(_m8(qa), C)], sem
        )
        yield None, pltpu.make_async_copy(
            sb.at[pl.ds(_m8(ea - C - qa), C)], out_hbm.at[pl.ds(_m8(ea - C), C)], sem
        )
        yield (ea - qa) - 2 * C == 8, pltpu.make_async_copy(
            sb.at[pl.ds(C, C)], out_hbm.at[pl.ds(_m8(qa + C), C)], sem
        )

    def run(copies, do_wait):
        for cond, desc in copies:
            act = desc.wait if do_wait else desc.start
            if cond is None:
                act()
            else:
                pl.when(cond)(act)

    def vpass(g, gb, sb):
        d_q, sh = g["d_q"], g["sh"]
        nv = (g["ea"] - g["qa"] + 15) >> 4
        idx0 = lanes + jnp.where(lanes < d_q, 8 - d_q, sh)
        sb[pl.ds(0, 16)] = plsc.load_gather(gb, [idx0])

        def vbody4(j, idx):
            s0 = pl.multiple_of(16 + 64 * j, 16)
            for q in range(4):
                sb[pl.ds(s0 + 16 * q, 16)] = plsc.load_gather(gb, [idx + 16 * q])
            return idx + 64

        lax.fori_loop(0, (nv + 2) >> 2, vbody4, lanes + 16 + sh)

    for k in range(5, 12):
        L_lo = max((1 << k) + 7, 65)
        L_hi = min((1 << (k + 1)) + 6, _N)
        if L_lo > L_hi:
            continue
        iA, iB = _N - L_hi, _N - L_lo
        C = 1 << k
        cnt = iB - iA + 1
        nt = (cnt - wid + _NW - 1) // _NW

        def row(j, iA=iA):
            return iA + wid + _NW * j

        def side(u, j, gb, sb, sg, ss, C=C, nt=nt, row=row):
            g = _geom(row(j))
            run(g_copies(g, C, gb, sg), do_wait=True)

            @pl.when(u >= 1)
            def _ws():
                run(s_copies(_geom(row(j - 2)), C, sb, ss), do_wait=True)

            vpass(g, gb, sb)
            run(s_copies(g, C, sb, ss), do_wait=False)

            @pl.when(j + 2 < nt)
            def _ig():
                run(g_copies(_geom(row(j + 2)), C, gb, sg), do_wait=False)

        @pl.when(nt > 0)
        def _p0(C=C, nt=nt, row=row):
            run(g_copies(_geom(row(0)), C, gb0, sgA), do_wait=False)

            @pl.when(nt > 1)
            def _p1():
                run(g_copies(_geom(row(1)), C, gb1, sgB), do_wait=False)

        def ubody(u, carry, C=C, nt=nt, row=row):
            side(u, 2 * u, gb0, sb0, sgA, ss0, C=C, nt=nt, row=row)

            @pl.when(2 * u + 1 < nt)
            def _b():
                side(u, 2 * u + 1, gb1, sb1, sgB, ss1, C=C, nt=nt, row=row)

            return carry

        lax.fori_loop(0, (nt + 1) >> 1, ubody, 0)

        @pl.when(nt > 0)
        def _e0(C=C, nt=nt, row=row):
            j0 = 2 * ((nt - 1) >> 1)
            run(s_copies(_geom(row(j0)), C, sb0, ss0), do_wait=True)

            @pl.when(nt > 1)
            def _e1():
                j1 = nt - 1 - (nt & 1)
                run(s_copies(_geom(row(j1)), C, sb1, ss1), do_wait=True)

    @pl.when(wid == _NW - 1)
    def _tail():
        def t_gathers(do_wait):
            descs = [
                pltpu.make_async_copy(
                    x_hbm.at[pl.ds(i * _N + _N - 64, 64)],
                    tbuf.at[pl.ds(64 * (i - _IC), 64)],
                    sgA,
                )
                for i in range(_IC, _N)
            ]
            descs.append(
                pltpu.make_async_copy(
                    x_hbm.at[pl.ds(_IC * _N - 8, 8)], tbuf.at[pl.ds(4096, 8)], sgA
                )
            )
            descs.append(pltpu.make_async_copy(tidx_hbm, ibuf, sgA))
            for d in descs:
                (d.wait if do_wait else d.start)()

        def t_scatters(do_wait):
            descs = [
                pltpu.make_async_copy(
                    sb0.at[pl.ds(256 * j, 256)],
                    out_hbm.at[pl.ds(_GT + 256 * j, 256)],
                    ss0,
                )
                for j in range(8)
            ]
            descs.append(
                pltpu.make_async_copy(
                    sb0.at[pl.ds(_TAIL_LEN - 256, 256)],
                    out_hbm.at[pl.ds(_T - 256, 256)],
                    ss0,
                )
            )
            for d in descs:
                (d.wait if do_wait else d.start)()

        t_gathers(False)
        t_gathers(True)

        def tvbody4(j, carry):
            s0 = pl.multiple_of(64 * j, 16)
            for q in range(4):
                idx = ibuf[pl.ds(s0 + 16 * q, 16)]
                sb0[pl.ds(s0 + 16 * q, 16)] = plsc.load_gather(tbuf, [idx])
            return carry

        lax.fori_loop(0, (((_TAIL_LEN + 15) >> 4) + 3) >> 2, tvbody4, 0)

        t_scatters(False)
        t_scatters(True)


_triu_call = functools.partial(
    pl.kernel,
    mesh=plsc.VectorSubcoreMesh(core_axis_name="c", subcore_axis_name="s"),
    out_type=jax.ShapeDtypeStruct((_T,), jnp.float32),
    compiler_params=pltpu.CompilerParams(needs_layout_passes=False),
    scratch_types=[
        pltpu.VMEM((_BUF,), jnp.float32),
        pltpu.VMEM((_BUF,), jnp.float32),
        pltpu.VMEM((_BUF,), jnp.float32),
        pltpu.VMEM((_BUF,), jnp.float32),
        pltpu.VMEM((_BUF,), jnp.float32),
        pltpu.VMEM((2112,), jnp.int32),
        pltpu.SemaphoreType.DMA,
        pltpu.SemaphoreType.DMA,
        pltpu.SemaphoreType.DMA,
        pltpu.SemaphoreType.DMA,
    ],
)(_triu_body)


def kernel(X):
    tidx = jnp.asarray(_TAIL_IDX)
    return _triu_call(X.reshape(-1), tidx)


# speedup vs baseline: 1.1480x; 1.1480x over previous
import functools

import jax
import jax.numpy as jnp
import numpy as np
from jax import lax
from jax.experimental import pallas as pl
from jax.experimental.pallas import tpu as pltpu
from jax.experimental.pallas import tpu_sc as plsc

_N = 4096
_T = _N * (_N + 1) // 2
_IC = _N - 64
_BUF = 4224
_NW = 32


def _off(i):
    return i * _N - (i * (i - 1)) // 2


def _tail_constants():
    off_ic = _off(_IC)
    d_qt = off_ic & 7
    gt = off_ic - d_qt
    tail_len = _T - gt
    idx = np.zeros(2112, dtype=np.int32)
    i = _IC
    for s in range(tail_len):
        p = gt + s
        if p < off_ic:
            idx[s] = 4096 + 8 - d_qt + s
        else:
            while _off(i + 1) <= p:
                i += 1
            idx[s] = 64 * (i - _IC) + (64 - (_N - i)) + (p - _off(i))
    return gt, tail_len, idx


_GT, _TAIL_LEN, _TAIL_IDX = _tail_constants()


def _m8(v):
    return pl.multiple_of(v, 8)


def _geom(i):
    L = _N - i
    f0 = i * _N + i
    d_s = f0 & 7
    off = i * _N - ((i * (i - 1)) >> 1)
    end = off + L
    d_q = off & 7
    d_e = end & 7
    return dict(
        i=i, L=L, d_s=d_s, fa0=f0 - d_s, off=off, end=end,
        d_q=d_q, qa=off - d_q, ea=end - d_e, sh=8 + d_s - d_q,
    )


def _triu_body(x_hbm, tidx_hbm, out_hbm, gb0, gb1, sb0, sb1, tbuf, ibuf,
               sgA, sgB, ss0, ss1):
    NC = 2
    wid = lax.axis_index("s") * NC + lax.axis_index("c")
    lanes = lax.iota(jnp.int32, 16)

    def g_copies(g, C, gb, sem):
        i = g["i"]
        yield None, pltpu.make_async_copy(
            x_hbm.at[pl.ds(_m8(lax.max(i * _N - 8, 0)), 8)], gb.at[pl.ds(0, 8)], sem
        )
        yield None, pltpu.make_async_copy(
            x_hbm.at[pl.ds(_m8(g["fa0"]), C)], gb.at[pl.ds(8, C)], sem
        )
        yield g["L"] + g["d_s"] > 2 * C, pltpu.make_async_copy(
            x_hbm.at[pl.ds(_m8(g["fa0"] + C), C)], gb.at[pl.ds(8 + C, C)], sem
        )
        yield None, pltpu.make_async_copy(
            x_hbm.at[pl.ds(_m8(i * _N + _N - C), C)],
            gb.at[pl.ds(_m8(8 + g["L"] + g["d_s"] - C), C)],
            sem,
        )

    def s_copies(g, C, sb, sem):
        qa, ea = g["qa"], g["ea"]
        yield None, pltpu.make_async_copy(
            sb.at[pl.ds(0, C)], out_hbm.at[pl.ds(_m8(qa), C)], sem
        )
        yield None, pltpu.make_async_copy(
            sb.at[pl.ds(_m8(ea - C - qa), C)], out_hbm.at[pl.ds(_m8(ea - C), C)], sem
        )
        yield (ea - qa) - 2 * C == 8, pltpu.make_async_copy(
            sb.at[pl.ds(C, C)], out_hbm.at[pl.ds(_m8(qa + C), C)], sem
        )

    def run(copies, do_wait):
        for cond, desc in copies:
            act = desc.wait if do_wait else desc.start
            if cond is None:
                act()
            else:
                pl.when(cond)(act)

    def vpass(g, gb, sb):
        d_q, sh = g["d_q"], g["sh"]
        nv = (g["ea"] - g["qa"] + 15) >> 4
        idx0 = lanes + jnp.where(lanes < d_q, 8 - d_q, sh)
        sb[pl.ds(0, 16)] = plsc.load_gather(gb, [idx0])

        def vbody4(j, idx):
            s0 = pl.multiple_of(16 + 64 * j, 16)
            vals = [plsc.load_gather(gb, [idx + 16 * q]) for q in range(4)]
            for q in range(4):
                sb[pl.ds(s0 + 16 * q, 16)] = vals[q]
            return idx + 64

        lax.fori_loop(0, (nv + 2) >> 2, vbody4, lanes + 16 + sh)

    for k in range(5, 12):
        L_lo = max((1 << k) + 7, 65)
        L_hi = min((1 << (k + 1)) + 6, _N)
        if L_lo > L_hi:
            continue
        iA, iB = _N - L_hi, _N - L_lo
        C = 1 << k
        cnt = iB - iA + 1
        nt = (cnt - wid + _NW - 1) // _NW

        def row(j, iA=iA):
            return iA + wid + _NW * j

        def side(u, j, gb, sb, sg, ss, C=C, nt=nt, row=row):
            g = _geom(row(j))
            run(g_copies(g, C, gb, sg), do_wait=True)

            @pl.when(u >= 1)
            def _ws():
                run(s_copies(_geom(row(j - 2)), C, sb, ss), do_wait=True)

            vpass(g, gb, sb)
            run(s_copies(g, C, sb, ss), do_wait=False)

            @pl.when(j + 2 < nt)
            def _ig():
                run(g_copies(_geom(row(j + 2)), C, gb, sg), do_wait=False)

        @pl.when(nt > 0)
        def _p0(C=C, nt=nt, row=row):
            run(g_copies(_geom(row(0)), C, gb0, sgA), do_wait=False)

            @pl.when(nt > 1)
            def _p1():
                run(g_copies(_geom(row(1)), C, gb1, sgB), do_wait=False)

        def ubody(u, carry, C=C, nt=nt, row=row):
            side(u, 2 * u, gb0, sb0, sgA, ss0, C=C, nt=nt, row=row)

            @pl.when(2 * u + 1 < nt)
            def _b():
                side(u, 2 * u + 1, gb1, sb1, sgB, ss1, C=C, nt=nt, row=row)

            return carry

        lax.fori_loop(0, (nt + 1) >> 1, ubody, 0)

        @pl.when(nt > 0)
        def _e0(C=C, nt=nt, row=row):
            j0 = 2 * ((nt - 1) >> 1)
            run(s_copies(_geom(row(j0)), C, sb0, ss0), do_wait=True)

            @pl.when(nt > 1)
            def _e1():
                j1 = nt - 1 - (nt & 1)
                run(s_copies(_geom(row(j1)), C, sb1, ss1), do_wait=True)

    @pl.when(wid == _NW - 1)
    def _tail():
        def t_gathers(do_wait):
            descs = [
                pltpu.make_async_copy(
                    x_hbm.at[pl.ds(i * _N + _N - 64, 64)],
                    tbuf.at[pl.ds(64 * (i - _IC), 64)],
                    sgA,
                )
                for i in range(_IC, _N)
            ]
            descs.append(
                pltpu.make_async_copy(
                    x_hbm.at[pl.ds(_IC * _N - 8, 8)], tbuf.at[pl.ds(4096, 8)], sgA
                )
            )
            descs.append(pltpu.make_async_copy(tidx_hbm, ibuf, sgA))
            for d in descs:
                (d.wait if do_wait else d.start)()

        def t_scatters(do_wait):
            descs = [
                pltpu.make_async_copy(
                    sb0.at[pl.ds(256 * j, 256)],
                    out_hbm.at[pl.ds(_GT + 256 * j, 256)],
                    ss0,
                )
                for j in range(8)
            ]
            descs.append(
                pltpu.make_async_copy(
                    sb0.at[pl.ds(_TAIL_LEN - 256, 256)],
                    out_hbm.at[pl.ds(_T - 256, 256)],
                    ss0,
                )
            )
            for d in descs:
                (d.wait if do_wait else d.start)()

        t_gathers(False)
        t_gathers(True)

        def tvbody4(j, carry):
            s0 = pl.multiple_of(64 * j, 16)
            for q in range(4):
                idx = ibuf[pl.ds(s0 + 16 * q, 16)]
                sb0[pl.ds(s0 + 16 * q, 16)] = plsc.load_gather(tbuf, [idx])
            return carry

        lax.fori_loop(0, (((_TAIL_LEN + 15) >> 4) + 3) >> 2, tvbody4, 0)

        t_scatters(False)
        t_scatters(True)


_triu_call = functools.partial(
    pl.kernel,
    mesh=plsc.VectorSubcoreMesh(core_axis_name="c", subcore_axis_name="s"),
    out_type=jax.ShapeDtypeStruct((_T,), jnp.float32),
    compiler_params=pltpu.CompilerParams(needs_layout_passes=False),
    scratch_types=[
        pltpu.VMEM((_BUF,), jnp.float32),
        pltpu.VMEM((_BUF,), jnp.float32),
        pltpu.VMEM((_BUF,), jnp.float32),
        pltpu.VMEM((_BUF,), jnp.float32),
        pltpu.VMEM((_BUF,), jnp.float32),
        pltpu.VMEM((2112,), jnp.int32),
        pltpu.SemaphoreType.DMA,
        pltpu.SemaphoreType.DMA,
        pltpu.SemaphoreType.DMA,
        pltpu.SemaphoreType.DMA,
    ],
)(_triu_body)


def kernel(X):
    tidx = jnp.asarray(_TAIL_IDX)
    return _triu_call(X.reshape(-1), tidx)


# speedup vs baseline: 1.1585x; 1.0091x over previous
import functools

import jax
import jax.numpy as jnp
import numpy as np
from jax import lax
from jax.experimental import pallas as pl
from jax.experimental.pallas import tpu as pltpu
from jax.experimental.pallas import tpu_sc as plsc

_N = 4096
_T = _N * (_N + 1) // 2
_IC = _N - 64
_BUF = 4288
_NW = 32


def _off(i):
    return i * _N - (i * (i - 1)) // 2


def _tail_constants():
    off_ic = _off(_IC)
    d_qt = off_ic & 7
    gt = off_ic - d_qt
    tail_len = _T - gt
    idx = np.zeros(2112, dtype=np.int32)
    i = _IC
    for s in range(tail_len):
        p = gt + s
        if p < off_ic:
            idx[s] = 4096 + 8 - d_qt + s
        else:
            while _off(i + 1) <= p:
                i += 1
            idx[s] = 64 * (i - _IC) + (64 - (_N - i)) + (p - _off(i))
    return gt, tail_len, idx


_GT, _TAIL_LEN, _TAIL_IDX = _tail_constants()


def _m8(v):
    return pl.multiple_of(v, 8)


def _geom(i):
    L = _N - i
    f0 = i * _N + i
    d_s = f0 & 7
    off = i * _N - ((i * (i - 1)) >> 1)
    end = off + L
    d_q = off & 7
    d_e = end & 7
    return dict(
        i=i, L=L, d_s=d_s, fa0=f0 - d_s, off=off, end=end,
        d_q=d_q, qa=off - d_q, ea=end - d_e, sh=8 + d_s - d_q,
    )


def _triu_body(x_hbm, tidx_hbm, out_hbm, gb0, gb1, sb0, sb1, tbuf, ibuf,
               sgA, sgB, ss0, ss1):
    NC = 2
    wid = lax.axis_index("s") * NC + lax.axis_index("c")
    lanes = lax.iota(jnp.int32, 16)

    def g_copies(g, C, gb, sem):
        i = g["i"]
        yield None, pltpu.make_async_copy(
            x_hbm.at[pl.ds(_m8(lax.max(i * _N - 8, 0)), 8)], gb.at[pl.ds(0, 8)], sem
        )
        yield None, pltpu.make_async_copy(
            x_hbm.at[pl.ds(_m8(g["fa0"]), C)], gb.at[pl.ds(8, C)], sem
        )
        yield g["L"] + g["d_s"] > 2 * C, pltpu.make_async_copy(
            x_hbm.at[pl.ds(_m8(g["fa0"] + C), C)], gb.at[pl.ds(8 + C, C)], sem
        )
        yield None, pltpu.make_async_copy(
            x_hbm.at[pl.ds(_m8(i * _N + _N - C), C)],
            gb.at[pl.ds(_m8(8 + g["L"] + g["d_s"] - C), C)],
            sem,
        )

    def s_copies(g, C, sb, sem):
        qa, ea = g["qa"], g["ea"]
        yield None, pltpu.make_async_copy(
            sb.at[pl.ds(0, C)], out_hbm.at[pl.ds(_m8(qa), C)], sem
        )
        yield None, pltpu.make_async_copy(
            sb.at[pl.ds(_m8(ea - C - qa), C)], out_hbm.at[pl.ds(_m8(ea - C), C)], sem
        )
        yield (ea - qa) - 2 * C == 8, pltpu.make_async_copy(
            sb.at[pl.ds(C, C)], out_hbm.at[pl.ds(_m8(qa + C), C)], sem
        )

    def run(copies, do_wait):
        for cond, desc in copies:
            act = desc.wait if do_wait else desc.start
            if cond is None:
                act()
            else:
                pl.when(cond)(act)

    def vpass(g, gb, sb):
        d_q, sh = g["d_q"], g["sh"]
        nv = (g["ea"] - g["qa"] + 15) >> 4
        idx0 = lanes + jnp.where(lanes < d_q, 8 - d_q, sh)
        sb[pl.ds(0, 16)] = plsc.load_gather(gb, [idx0])

        def vbody8(j, idx):
            s0 = pl.multiple_of(16 + 128 * j, 16)
            vals = [plsc.load_gather(gb, [idx + 16 * q]) for q in range(8)]
            for q in range(8):
                sb[pl.ds(s0 + 16 * q, 16)] = vals[q]
            return idx + 128

        lax.fori_loop(0, (nv + 6) >> 3, vbody8, lanes + 16 + sh)

    for k in range(5, 12):
        L_lo = max((1 << k) + 7, 65)
        L_hi = min((1 << (k + 1)) + 6, _N)
        if L_lo > L_hi:
            continue
        iA, iB = _N - L_hi, _N - L_lo
        C = 1 << k
        cnt = iB - iA + 1
        nt = (cnt - wid + _NW - 1) // _NW

        def row(j, iA=iA):
            return iA + wid + _NW * j

        def side(u, j, gb, sb, sg, ss, C=C, nt=nt, row=row):
            g = _geom(row(j))
            run(g_copies(g, C, gb, sg), do_wait=True)

            @pl.when(u >= 1)
            def _ws():
                run(s_copies(_geom(row(j - 2)), C, sb, ss), do_wait=True)

            vpass(g, gb, sb)
            run(s_copies(g, C, sb, ss), do_wait=False)

            @pl.when(j + 2 < nt)
            def _ig():
                run(g_copies(_geom(row(j + 2)), C, gb, sg), do_wait=False)

        @pl.when(nt > 0)
        def _p0(C=C, nt=nt, row=row):
            run(g_copies(_geom(row(0)), C, gb0, sgA), do_wait=False)

            @pl.when(nt > 1)
            def _p1():
                run(g_copies(_geom(row(1)), C, gb1, sgB), do_wait=False)

        def ubody(u, carry, C=C, nt=nt, row=row):
            side(u, 2 * u, gb0, sb0, sgA, ss0, C=C, nt=nt, row=row)

            @pl.when(2 * u + 1 < nt)
            def _b():
                side(u, 2 * u + 1, gb1, sb1, sgB, ss1, C=C, nt=nt, row=row)

            return carry

        lax.fori_loop(0, (nt + 1) >> 1, ubody, 0)

        @pl.when(nt > 0)
        def _e0(C=C, nt=nt, row=row):
            j0 = 2 * ((nt - 1) >> 1)
            run(s_copies(_geom(row(j0)), C, sb0, ss0), do_wait=True)

            @pl.when(nt > 1)
            def _e1():
                j1 = nt - 1 - (nt & 1)
                run(s_copies(_geom(row(j1)), C, sb1, ss1), do_wait=True)

    @pl.when(wid == _NW - 1)
    def _tail():
        def t_gathers(do_wait):
            descs = [
                pltpu.make_async_copy(
                    x_hbm.at[pl.ds(i * _N + _N - 64, 64)],
                    tbuf.at[pl.ds(64 * (i - _IC), 64)],
                    sgA,
                )
                for i in range(_IC, _N)
            ]
            descs.append(
                pltpu.make_async_copy(
                    x_hbm.at[pl.ds(_IC * _N - 8, 8)], tbuf.at[pl.ds(4096, 8)], sgA
                )
            )
            descs.append(pltpu.make_async_copy(tidx_hbm, ibuf, sgA))
            for d in descs:
                (d.wait if do_wait else d.start)()

        def t_scatters(do_wait):
            descs = [
                pltpu.make_async_copy(
                    sb0.at[pl.ds(256 * j, 256)],
                    out_hbm.at[pl.ds(_GT + 256 * j, 256)],
                    ss0,
                )
                for j in range(8)
            ]
            descs.append(
                pltpu.make_async_copy(
                    sb0.at[pl.ds(_TAIL_LEN - 256, 256)],
                    out_hbm.at[pl.ds(_T - 256, 256)],
                    ss0,
                )
            )
            for d in descs:
                (d.wait if do_wait else d.start)()

        t_gathers(False)
        t_gathers(True)

        def tvbody4(j, carry):
            s0 = pl.multiple_of(64 * j, 16)
            for q in range(4):
                idx = ibuf[pl.ds(s0 + 16 * q, 16)]
                sb0[pl.ds(s0 + 16 * q, 16)] = plsc.load_gather(tbuf, [idx])
            return carry

        lax.fori_loop(0, (((_TAIL_LEN + 15) >> 4) + 3) >> 2, tvbody4, 0)

        t_scatters(False)
        t_scatters(True)


_triu_call = functools.partial(
    pl.kernel,
    mesh=plsc.VectorSubcoreMesh(core_axis_name="c", subcore_axis_name="s"),
    out_type=jax.ShapeDtypeStruct((_T,), jnp.float32),
    compiler_params=pltpu.CompilerParams(needs_layout_passes=False),
    scratch_types=[
        pltpu.VMEM((_BUF,), jnp.float32),
        pltpu.VMEM((_BUF,), jnp.float32),
        pltpu.VMEM((_BUF,), jnp.float32),
        pltpu.VMEM((_BUF,), jnp.float32),
        pltpu.VMEM((_BUF,), jnp.float32),
        pltpu.VMEM((2112,), jnp.int32),
        pltpu.SemaphoreType.DMA,
        pltpu.SemaphoreType.DMA,
        pltpu.SemaphoreType.DMA,
        pltpu.SemaphoreType.DMA,
    ],
)(_triu_body)


def kernel(X):
    tidx = jnp.asarray(_TAIL_IDX)
    return _triu_call(X.reshape(-1), tidx)
